# baseline (speedup 1.0000x reference)
import functools

import jax
import jax.numpy as jnp
from jax import lax
from jax.experimental import pallas as pl
from jax.experimental.pallas import tpu as pltpu
from jax.experimental.pallas import tpu_sc as plsc

BATCH = 16384
NUM_ROWS = 1000000
EMBED_DIM = 32
FEAT = 16
NUM_CORES = 2
NUM_SUBCORES = 16
NUM_WORKERS = NUM_CORES * NUM_SUBCORES
BPW = BATCH // NUM_WORKERS
LANES = 16
TC_BLOCK = 49152


def _tc_project(utab_t, itab_t, wu_bcast, wi_bcast):
    def body(wu_ref, wi_ref, ut_ref, it_ref, ou_ref, oi_ref):
        ou_ref[...] = jnp.sum(ut_ref[...] * wu_ref[:, 0:1], axis=0)
        oi_ref[...] = jnp.sum(it_ref[...] * wi_ref[:, 0:1], axis=0)

    grid = pl.cdiv(NUM_ROWS, TC_BLOCK)
    return pl.pallas_call(
        body,
        grid=(grid,),
        in_specs=[
            pl.BlockSpec((EMBED_DIM, 128), lambda i: (0, 0)),
            pl.BlockSpec((EMBED_DIM, 128), lambda i: (0, 0)),
            pl.BlockSpec((EMBED_DIM, TC_BLOCK), lambda i: (0, i)),
            pl.BlockSpec((EMBED_DIM, TC_BLOCK), lambda i: (0, i)),
        ],
        out_specs=[
            pl.BlockSpec((TC_BLOCK,), lambda i: (i,)),
            pl.BlockSpec((TC_BLOCK,), lambda i: (i,)),
        ],
        out_shape=[
            jax.ShapeDtypeStruct((NUM_ROWS,), jnp.float32),
            jax.ShapeDtypeStruct((NUM_ROWS,), jnp.float32),
        ],
    )(wu_bcast, wi_bcast, utab_t, itab_t)


def _sc_combine(user, item, feat_t, proj_u, proj_i, w_flat, fcb16, gb16):
    mesh = plsc.VectorSubcoreMesh(core_axis_name="c", subcore_axis_name="s")

    @functools.partial(
        pl.kernel,
        out_type=jax.ShapeDtypeStruct((BATCH,), jnp.float32),
        mesh=mesh,
        compiler_params=pltpu.CompilerParams(needs_layout_passes=False),
        scratch_types=[
            pltpu.VMEM((BPW,), jnp.int32),
            pltpu.VMEM((BPW,), jnp.int32),
            pltpu.VMEM((BPW,), jnp.float32),
            pltpu.VMEM((BPW,), jnp.float32),
            pltpu.VMEM((FEAT, BPW), jnp.float32),
            pltpu.VMEM((80,), jnp.float32),
            pltpu.VMEM((LANES,), jnp.float32),
            pltpu.VMEM((LANES,), jnp.float32),
            pltpu.VMEM((BPW,), jnp.float32),
            pltpu.SemaphoreType.DMA,
            pltpu.SemaphoreType.DMA,
        ],
    )
    def body(user_h, item_h, feat_h, pju_h, pji_h, w_h, fcb_h, gb_h,
             out_h, uidx, iidx, pu, pi, fslab, wv, fcbv, gbv, outv,
             usem, isem):
        wid = lax.axis_index("s") * NUM_CORES + lax.axis_index("c")
        base = pl.multiple_of(wid * BPW, BPW)
        pltpu.sync_copy(user_h.at[pl.ds(base, BPW)], uidx)
        pltpu.sync_copy(item_h.at[pl.ds(base, BPW)], iidx)
        cu = pltpu.async_copy(pju_h.at[uidx], pu, usem)
        ci = pltpu.async_copy(pji_h.at[iidx], pi, isem)
        pltpu.sync_copy(feat_h.at[:, pl.ds(base, BPW)], fslab)
        pltpu.sync_copy(w_h, wv)
        pltpu.sync_copy(fcb_h, fcbv)
        pltpu.sync_copy(gb_h, gbv)
        wfv = wv[pl.ds(64, LANES)]
        biasv = fcbv[...] + gbv[...]
        cu.wait()
        ci.wait()

        def chunk(g, carry):
            b0 = g * LANES
            acc = pu[pl.ds(b0, LANES)] + pi[pl.ds(b0, LANES)] + biasv
            for f in range(FEAT):
                acc = acc + fslab[f, pl.ds(b0, LANES)] * wfv[f]
            outv[pl.ds(b0, LANES)] = acc
            return carry

        lax.fori_loop(0, BPW // LANES, chunk, 0)
        pltpu.sync_copy(outv, out_h.at[pl.ds(base, BPW)])

    return body(user, item, feat_t, proj_u, proj_i, w_flat, fcb16, gb16)


def kernel(user, item, item_feature, user_table, item_table, fc_w, fc_b,
           global_bias):
    w_flat = fc_w.reshape(-1)
    wu_bcast = jnp.broadcast_to(w_flat[:EMBED_DIM, None], (EMBED_DIM, 128))
    wi_bcast = jnp.broadcast_to(
        w_flat[EMBED_DIM:2 * EMBED_DIM, None], (EMBED_DIM, 128))
    fcb16 = jnp.broadcast_to(fc_b, (LANES,))
    gb16 = jnp.broadcast_to(global_bias, (LANES,))
    proj_u, proj_i = _tc_project(user_table.T, item_table.T,
                                 wu_bcast, wi_bcast)
    out = _sc_combine(user, item, item_feature.T, proj_u, proj_i,
                      w_flat, fcb16, gb16)
    return out.reshape(BATCH, 1)


# speedup vs baseline: 1.0061x; 1.0061x over previous
import functools

import jax
import jax.numpy as jnp
from jax import lax
from jax.experimental import pallas as pl
from jax.experimental.pallas import tpu as pltpu
from jax.experimental.pallas import tpu_sc as plsc

BATCH = 16384
NUM_ROWS = 1000000
EMBED_DIM = 32
FEAT = 16
NUM_CORES = 2
NUM_SUBCORES = 16
NUM_WORKERS = NUM_CORES * NUM_SUBCORES
BPW = BATCH // NUM_WORKERS
LANES = 16
TC_BLOCK = 40960


def _tc_project(utab_t, itab_t, wu_bcast, wi_bcast):
    def body(wu_ref, wi_ref, ut_ref, it_ref, ou_ref, oi_ref):
        ou_ref[...] = jnp.sum(ut_ref[...] * wu_ref[:, 0:1], axis=0)
        oi_ref[...] = jnp.sum(it_ref[...] * wi_ref[:, 0:1], axis=0)

    grid = pl.cdiv(NUM_ROWS, TC_BLOCK)
    return pl.pallas_call(
        body,
        grid=(grid,),
        in_specs=[
            pl.BlockSpec((EMBED_DIM, 128), lambda i: (0, 0)),
            pl.BlockSpec((EMBED_DIM, 128), lambda i: (0, 0)),
            pl.BlockSpec((EMBED_DIM, TC_BLOCK), lambda i: (0, i)),
            pl.BlockSpec((EMBED_DIM, TC_BLOCK), lambda i: (0, i)),
        ],
        out_specs=[
            pl.BlockSpec((TC_BLOCK,), lambda i: (i,)),
            pl.BlockSpec((TC_BLOCK,), lambda i: (i,)),
        ],
        out_shape=[
            jax.ShapeDtypeStruct((NUM_ROWS,), jnp.float32),
            jax.ShapeDtypeStruct((NUM_ROWS,), jnp.float32),
        ],
    )(wu_bcast, wi_bcast, utab_t, itab_t)


def _sc_combine(user, item, feat_t, proj_u, proj_i, w_flat, fcb16, gb16):
    mesh = plsc.VectorSubcoreMesh(core_axis_name="c", subcore_axis_name="s")

    @functools.partial(
        pl.kernel,
        out_type=jax.ShapeDtypeStruct((BATCH,), jnp.float32),
        mesh=mesh,
        compiler_params=pltpu.CompilerParams(needs_layout_passes=False),
        scratch_types=[
            pltpu.VMEM((BPW,), jnp.int32),
            pltpu.VMEM((BPW,), jnp.int32),
            pltpu.VMEM((BPW,), jnp.float32),
            pltpu.VMEM((BPW,), jnp.float32),
            pltpu.VMEM((FEAT, BPW), jnp.float32),
            pltpu.VMEM((80,), jnp.float32),
            pltpu.VMEM((LANES,), jnp.float32),
            pltpu.VMEM((LANES,), jnp.float32),
            pltpu.VMEM((BPW,), jnp.float32),
            pltpu.SemaphoreType.DMA,
            pltpu.SemaphoreType.DMA,
        ],
    )
    def body(user_h, item_h, feat_h, pju_h, pji_h, w_h, fcb_h, gb_h,
             out_h, uidx, iidx, pu, pi, fslab, wv, fcbv, gbv, outv,
             usem, isem):
        wid = lax.axis_index("s") * NUM_CORES + lax.axis_index("c")
        base = pl.multiple_of(wid * BPW, BPW)
        pltpu.sync_copy(user_h.at[pl.ds(base, BPW)], uidx)
        pltpu.sync_copy(item_h.at[pl.ds(base, BPW)], iidx)
        cu = pltpu.async_copy(pju_h.at[uidx], pu, usem)
        ci = pltpu.async_copy(pji_h.at[iidx], pi, isem)
        pltpu.sync_copy(feat_h.at[:, pl.ds(base, BPW)], fslab)
        pltpu.sync_copy(w_h, wv)
        pltpu.sync_copy(fcb_h, fcbv)
        pltpu.sync_copy(gb_h, gbv)
        wfv = wv[pl.ds(64, LANES)]
        biasv = fcbv[...] + gbv[...]
        cu.wait()
        ci.wait()

        def chunk(g, carry):
            b0 = g * LANES
            acc = pu[pl.ds(b0, LANES)] + pi[pl.ds(b0, LANES)] + biasv
            for f in range(FEAT):
                acc = acc + fslab[f, pl.ds(b0, LANES)] * wfv[f]
            outv[pl.ds(b0, LANES)] = acc
            return carry

        lax.fori_loop(0, BPW // LANES, chunk, 0)
        pltpu.sync_copy(outv, out_h.at[pl.ds(base, BPW)])

    return body(user, item, feat_t, proj_u, proj_i, w_flat, fcb16, gb16)


def kernel(user, item, item_feature, user_table, item_table, fc_w, fc_b,
           global_bias):
    w_flat = fc_w.reshape(-1)
    wu_bcast = jnp.broadcast_to(w_flat[:EMBED_DIM, None], (EMBED_DIM, 128))
    wi_bcast = jnp.broadcast_to(
        w_flat[EMBED_DIM:2 * EMBED_DIM, None], (EMBED_DIM, 128))
    fcb16 = jnp.broadcast_to(fc_b, (LANES,))
    gb16 = jnp.broadcast_to(global_bias, (LANES,))
    proj_u, proj_i = _tc_project(user_table.T, item_table.T,
                                 wu_bcast, wi_bcast)
    out = _sc_combine(user, item, item_feature.T, proj_u, proj_i,
                      w_flat, fcb16, gb16)
    return out.reshape(BATCH, 1)
